# Initial kernel scaffold; baseline (speedup 1.0000x reference)
import jax
import jax.numpy as jnp
from jax.experimental import pallas as pl


def kernel(hidden_states, W):
    raise NotImplementedError("write your pallas kernel here")


# speedup vs baseline: 1.2284x; 1.2284x over previous
import functools

import jax
import jax.numpy as jnp
from jax.experimental import pallas as pl

NUM_EXPERTS = 64
TOP_K = 8
BLK = 1024


def _router_block(x_ref, w_ref, idx_ref, val_ref):
    x = x_ref[...]
    w = w_ref[...]
    logits = jax.lax.dot_general(
        x, w,
        dimension_numbers=(((1,), (1,)), ((), ())),
        preferred_element_type=jnp.float32,
    )

    blk = logits.shape[0]
    lane_e = jax.lax.broadcasted_iota(jnp.int32, logits.shape, 1)
    lane_k = jax.lax.broadcasted_iota(jnp.int32, (blk, TOP_K), 1)

    cur = logits
    idx_out = jnp.zeros((blk, TOP_K), jnp.int32)
    val_out = jnp.zeros((blk, TOP_K), jnp.float32)
    for k in range(TOP_K):
        m = jnp.max(cur, axis=1, keepdims=True)
        sel = jnp.min(jnp.where(cur == m, lane_e, NUM_EXPERTS),
                      axis=1, keepdims=True)
        idx_out = jnp.where(lane_k == k, sel, idx_out)
        val_out = jnp.where(lane_k == k, m, val_out)
        cur = jnp.where(lane_e == sel, -jnp.inf, cur)

    e = jnp.exp(val_out - val_out[:, :1])
    val_ref[...] = e / jnp.sum(e, axis=1, keepdims=True)
    idx_ref[...] = idx_out


@functools.partial(jax.jit, static_argnames=())
def kernel(hidden_states, W):
    b, s, d = hidden_states.shape
    n = b * s
    flat = hidden_states.reshape(n, d)
    grid = (n // BLK,)
    out = pl.pallas_call(
        _router_block,
        grid=grid,
        in_specs=[
            pl.BlockSpec((BLK, d), lambda i: (i, 0)),
            pl.BlockSpec((NUM_EXPERTS, d), lambda i: (0, 0)),
        ],
        out_specs=[
            pl.BlockSpec((BLK, TOP_K), lambda i: (i, 0)),
            pl.BlockSpec((BLK, TOP_K), lambda i: (i, 0)),
        ],
        out_shape=[
            jax.ShapeDtypeStruct((n, TOP_K), jnp.int32),
            jax.ShapeDtypeStruct((n, TOP_K), jnp.float32),
        ],
    )(flat, W)
    return out[0], out[1]


# speedup vs baseline: 1.8245x; 1.4852x over previous
import functools

import jax
import jax.numpy as jnp
from jax.experimental import pallas as pl

NUM_EXPERTS = 64
TOP_K = 8
BLK = 1024


def _router_block(x_ref, w_ref, idx_ref, val_ref):
    x = x_ref[...]
    w = w_ref[...]
    logits = jax.lax.dot_general(
        x, w,
        dimension_numbers=(((1,), (1,)), ((), ())),
        preferred_element_type=jnp.float32,
    )

    blk = logits.shape[0]
    lt = logits.T
    sub_e = jax.lax.broadcasted_iota(jnp.int32, (NUM_EXPERTS, blk), 0)
    sub_k = jax.lax.broadcasted_iota(jnp.int32, (TOP_K, blk), 0)

    idx_out = jnp.zeros((TOP_K, blk), jnp.int32)
    val_out = jnp.zeros((TOP_K, blk), jnp.float32)
    for k in range(TOP_K):
        m = jnp.max(lt, axis=0, keepdims=True)
        sel = jnp.min(jnp.where(lt == m, sub_e, NUM_EXPERTS),
                      axis=0, keepdims=True)
        idx_out = jnp.where(sub_k == k, sel, idx_out)
        val_out = jnp.where(sub_k == k, m, val_out)
        lt = jnp.where(sub_e == sel, -jnp.inf, lt)

    e = jnp.exp(val_out - val_out[:1])
    w_out = e / jnp.sum(e, axis=0, keepdims=True)
    idx_ref[...] = idx_out.T
    val_ref[...] = w_out.T


@functools.partial(jax.jit, static_argnames=())
def kernel(hidden_states, W):
    b, s, d = hidden_states.shape
    n = b * s
    flat = hidden_states.reshape(n, d)
    grid = (n // BLK,)
    out = pl.pallas_call(
        _router_block,
        grid=grid,
        in_specs=[
            pl.BlockSpec((BLK, d), lambda i: (i, 0)),
            pl.BlockSpec((NUM_EXPERTS, d), lambda i: (0, 0)),
        ],
        out_specs=[
            pl.BlockSpec((BLK, TOP_K), lambda i: (i, 0)),
            pl.BlockSpec((BLK, TOP_K), lambda i: (i, 0)),
        ],
        out_shape=[
            jax.ShapeDtypeStruct((n, TOP_K), jnp.int32),
            jax.ShapeDtypeStruct((n, TOP_K), jnp.float32),
        ],
    )(flat, W)
    return out[0], out[1]


# speedup vs baseline: 1.8278x; 1.0018x over previous
import functools

import jax
import jax.numpy as jnp
from jax.experimental import pallas as pl
from jax.experimental.pallas import tpu as pltpu

NUM_EXPERTS = 64
TOP_K = 8
BLK = 1024


def _router_block(x_ref, w_ref, idx_ref, val_ref):
    x = x_ref[...]
    w = w_ref[...]
    logits = jax.lax.dot_general(
        x, w,
        dimension_numbers=(((1,), (1,)), ((), ())),
        preferred_element_type=jnp.float32,
    )

    blk = logits.shape[0]
    lt = logits.T
    sub_e = jax.lax.broadcasted_iota(jnp.int32, (NUM_EXPERTS, blk), 0)
    sub_k = jax.lax.broadcasted_iota(jnp.int32, (TOP_K, blk), 0)

    idx_out = jnp.zeros((TOP_K, blk), jnp.int32)
    val_out = jnp.zeros((TOP_K, blk), jnp.float32)
    for k in range(TOP_K):
        m = jnp.max(lt, axis=0, keepdims=True)
        sel = jnp.min(jnp.where(lt == m, sub_e, NUM_EXPERTS),
                      axis=0, keepdims=True)
        idx_out = jnp.where(sub_k == k, sel, idx_out)
        val_out = jnp.where(sub_k == k, m, val_out)
        lt = jnp.where(sub_e == sel, -jnp.inf, lt)

    e = jnp.exp(val_out - val_out[:1])
    w_out = e / jnp.sum(e, axis=0, keepdims=True)
    idx_ref[...] = idx_out.T
    val_ref[...] = w_out.T


@functools.partial(jax.jit, static_argnames=())
def kernel(hidden_states, W):
    b, s, d = hidden_states.shape
    n = b * s
    flat = hidden_states.reshape(n, d)
    grid = (n // BLK,)
    out = pl.pallas_call(
        _router_block,
        grid=grid,
        in_specs=[
            pl.BlockSpec((BLK, d), lambda i: (i, 0)),
            pl.BlockSpec((NUM_EXPERTS, d), lambda i: (0, 0)),
        ],
        out_specs=[
            pl.BlockSpec((BLK, TOP_K), lambda i: (i, 0)),
            pl.BlockSpec((BLK, TOP_K), lambda i: (i, 0)),
        ],
        out_shape=[
            jax.ShapeDtypeStruct((n, TOP_K), jnp.int32),
            jax.ShapeDtypeStruct((n, TOP_K), jnp.float32),
        ],
        compiler_params=pltpu.CompilerParams(
            dimension_semantics=("parallel",),
        ),
    )(flat, W)
    return out[0], out[1]


# speedup vs baseline: 1.8925x; 1.0354x over previous
import functools

import jax
import jax.numpy as jnp
from jax.experimental import pallas as pl
from jax.experimental.pallas import tpu as pltpu

NUM_EXPERTS = 64
TOP_K = 8
BLK = 2048


def _router_block(x_ref, w_ref, idx_ref, val_ref):
    x = x_ref[...]
    w = w_ref[...]
    logits = jax.lax.dot_general(
        x, w,
        dimension_numbers=(((1,), (1,)), ((), ())),
        preferred_element_type=jnp.float32,
    )

    blk = logits.shape[0]
    lt = logits.T
    sub_e = jax.lax.broadcasted_iota(jnp.int32, (NUM_EXPERTS, blk), 0)
    sub_k = jax.lax.broadcasted_iota(jnp.int32, (TOP_K, blk), 0)

    idx_out = jnp.zeros((TOP_K, blk), jnp.int32)
    val_out = jnp.zeros((TOP_K, blk), jnp.float32)
    for k in range(TOP_K):
        m = jnp.max(lt, axis=0, keepdims=True)
        sel = jnp.min(jnp.where(lt == m, sub_e, NUM_EXPERTS),
                      axis=0, keepdims=True)
        idx_out = jnp.where(sub_k == k, sel, idx_out)
        val_out = jnp.where(sub_k == k, m, val_out)
        lt = jnp.where(sub_e == sel, -jnp.inf, lt)

    e = jnp.exp(val_out - val_out[:1])
    w_out = e / jnp.sum(e, axis=0, keepdims=True)
    idx_ref[...] = idx_out.T
    val_ref[...] = w_out.T


@functools.partial(jax.jit, static_argnames=())
def kernel(hidden_states, W):
    b, s, d = hidden_states.shape
    n = b * s
    flat = hidden_states.reshape(n, d)
    grid = (n // BLK,)
    out = pl.pallas_call(
        _router_block,
        grid=grid,
        in_specs=[
            pl.BlockSpec((BLK, d), lambda i: (i, 0)),
            pl.BlockSpec((NUM_EXPERTS, d), lambda i: (0, 0)),
        ],
        out_specs=[
            pl.BlockSpec((BLK, TOP_K), lambda i: (i, 0)),
            pl.BlockSpec((BLK, TOP_K), lambda i: (i, 0)),
        ],
        out_shape=[
            jax.ShapeDtypeStruct((n, TOP_K), jnp.int32),
            jax.ShapeDtypeStruct((n, TOP_K), jnp.float32),
        ],
        compiler_params=pltpu.CompilerParams(
            dimension_semantics=("parallel",),
        ),
    )(flat, W)
    return out[0], out[1]
